# baseline (speedup 1.0000x reference)
import jax
import jax.numpy as jnp
from jax.experimental import pallas as pl
from jax.experimental.pallas import tpu as pltpu

ALPHA = 0.1
ROW_BLOCK = 512


def _gconv_block(a_ref, xtt_ref, x0t_ref, mk_ref, bt_ref, out_ref):
    a_bf = a_ref[...].astype(jnp.bfloat16)
    agg = jax.lax.dot_general(
        a_bf, xtt_ref[...], (((1,), (1,)), ((), ())),
        preferred_element_type=jnp.float32)
    h = (1.0 - ALPHA) * agg + ALPHA * x0t_ref[...]
    hw = jax.lax.dot_general(
        h.astype(jnp.bfloat16), mk_ref[...], (((1,), (1,)), ((), ())),
        preferred_element_type=jnp.float32)
    out_ref[...] = jax.nn.gelu(hw + bt_ref[...])


def kernel(x, x0, adj, W, b):
    B, N, D = x.shape
    BD = B * D
    xtt = jnp.transpose(x, (0, 2, 1)).reshape(BD, N).astype(jnp.bfloat16)
    x0t = jnp.transpose(x0, (1, 0, 2)).reshape(N, BD)
    m = 0.5 * (jnp.eye(D, dtype=jnp.float32) + W)
    mk = jnp.kron(jnp.eye(B, dtype=jnp.float32), m.T).astype(jnp.bfloat16)
    bt = jnp.tile(b, B).reshape(1, BD)

    grid = (N // ROW_BLOCK,)
    outt = pl.pallas_call(
        _gconv_block,
        grid=grid,
        in_specs=[
            pl.BlockSpec((ROW_BLOCK, N), lambda i: (i, 0)),
            pl.BlockSpec((BD, N), lambda i: (0, 0)),
            pl.BlockSpec((ROW_BLOCK, BD), lambda i: (i, 0)),
            pl.BlockSpec((BD, BD), lambda i: (0, 0)),
            pl.BlockSpec((1, BD), lambda i: (0, 0)),
        ],
        out_specs=pl.BlockSpec((ROW_BLOCK, BD), lambda i: (i, 0)),
        out_shape=jax.ShapeDtypeStruct((N, BD), jnp.float32),
        compiler_params=pltpu.CompilerParams(
            dimension_semantics=("parallel",),
        ),
    )(adj, xtt, x0t, mk, bt)
    return jnp.transpose(outt.reshape(N, B, D), (1, 0, 2))
